# baseline (speedup 1.0000x reference)
import numpy as np
import jax
import jax.numpy as jnp
from jax.experimental import pallas as pl

KSIZE = 3
NSIZE = 7
H = 224
HS = 56
C = 3
C1 = C * KSIZE * KSIZE
PAD2 = NSIZE // 2

_HIGH = jax.lax.Precision.HIGHEST


def _resize_matrix(in_size, out_size):
    scale = out_size / in_size
    inv_scale = 1.0 / scale
    kernel_scale = max(inv_scale, 1.0)
    sample_f = (np.arange(out_size, dtype=np.float32) + 0.5) * inv_scale - 0.5
    x = np.abs(sample_f[None, :]
               - np.arange(in_size, dtype=np.float32)[:, None]) / kernel_scale
    x = x.astype(np.float32)
    w = (((1.5 * x - 2.5) * x * x + 1.0) * (x <= 1.0)
         + ((((-0.5 * x + 2.5) * x - 4.0) * x + 2.0)
            * ((x > 1.0) & (x < 2.0)))).astype(np.float32)
    total = w.sum(axis=0, keepdims=True)
    w = np.where(np.abs(total) > 1000 * np.finfo(np.float32).eps,
                 w / np.where(total != 0, total, 1), 0)
    w = np.where(((sample_f >= -0.5) & (sample_f <= in_size - 0.5))[None, :],
                 w, 0)
    return np.ascontiguousarray(w.T.astype(np.float32))


_RESIZE_W = _resize_matrix(H, HS)


def _refl(i, n):
    if i < 0:
        return -i
    if i >= n:
        return 2 * n - 2 - i
    return i


def _sel_matrices():
    spj = np.zeros((KSIZE, HS, HS), np.float32)
    for i in range(KSIZE):
        for y in range(HS):
            spj[i, y, _refl(y + i - 1, HS)] = 1.0
    HP = HS + 2 * PAD2
    snj = np.zeros((KSIZE, HP, HS), np.float32)
    for i in range(KSIZE):
        for Y in range(HP):
            snj[i, Y, _refl(_refl(Y - PAD2, HS) + i - 1, HS)] = 1.0
    spy = spj.reshape(KSIZE * HS, HS)
    sny = snj.reshape(KSIZE * HP, HS)
    return spy, spj, sny, snj


_SPY, _SPJ, _SNY, _SNJ = _sel_matrices()


def _fused_body(xpred_ref, xi_ref, w_ref, spy_ref, spj_ref, sny_ref, snj_ref,
                out_ref):
    w = w_ref[...]

    def census(xp):
        x = xp[:, 1:1 + H, 1:1 + H]
        acc = jnp.zeros((C, H, H), jnp.float32)
        for i in range(KSIZE):
            for j in range(KSIZE):
                acc = acc + jnp.tanh(xp[:, i:i + H, j:j + H] - x)
        return acc * (1.0 / (KSIZE * KSIZE))

    def resize(m):
        t1 = jax.lax.dot_general(m, w, (((1,), (1,)), ((), ())),
                                 precision=_HIGH)
        return jax.lax.dot_general(t1, w, (((1,), (1,)), ((), ())),
                                   precision=_HIGH)

    def _rpad(x, p, axis):
        n = x.shape[axis]

        def sl(a, bnd):
            return tuple(slice(a, bnd) if d == axis else slice(None)
                         for d in range(x.ndim))

        parts = ([x[sl(t, t + 1)] for t in range(p, 0, -1)]
                 + [x]
                 + [x[sl(n - 1 - t, n - t)] for t in range(1, p + 1)])
        return jnp.concatenate(parts, axis=axis)

    def patches(r):
        rp = _rpad(_rpad(r, 1, 1), 1, 2)
        cols = [rp[:, i:i + HS, j:j + HS]
                for i in range(KSIZE) for j in range(KSIZE)]
        return jnp.stack(cols, axis=1).reshape(C1, HS, HS)

    maps = [xpred_ref[0], xi_ref[0, 0], xi_ref[0, 1]]
    rz_ct = [resize(census(m)) for m in maps]
    rz_raw = [resize(m[:, 1:1 + H, 1:1 + H]) for m in maps]

    pct = patches(rz_ct[0])
    praw = patches(rz_raw[0])
    nct = [_rpad(_rpad(patches(rz_ct[k]), PAD2, 1), PAD2, 2) for k in (1, 2)]
    nraw = [_rpad(_rpad(patches(rz_raw[k]), PAD2, 1), PAD2, 2) for k in (1, 2)]

    HP = HS + 2 * PAD2
    npc = jnp.concatenate(nct, axis=2)
    npr = jnp.concatenate(nraw, axis=2)
    z6 = jnp.zeros((C1, HS, HP - HS), jnp.float32)
    ppc = jnp.concatenate([pct, z6, pct, z6], axis=2)
    ppr = jnp.concatenate([praw, z6, praw, z6], axis=2)
    z6s = jnp.zeros((HS, HP - HS), jnp.float32)
    pss_c = jnp.sum(pct * pct, axis=0)
    pss_r = jnp.sum(praw * praw, axis=0)
    pssp_c = jnp.concatenate([pss_c, z6s, pss_c, z6s], axis=1)
    pssp_r = jnp.concatenate([pss_r, z6s, pss_r, z6s], axis=1)
    nss_c = jnp.sum(npc * npc, axis=0)
    nss_r = jnp.sum(npr * npr, axis=0)

    best_d = None
    best_raw = None
    for dx in range(NSIZE):
        ppc_dx = jnp.roll(ppc, dx, axis=2) if dx else ppc
        ppr_dx = jnp.roll(ppr, dx, axis=2) if dx else ppr
        def unroll(x):
            return jnp.roll(x, -dx, axis=1) if dx else x

        for dy in range(NSIZE):
            cross_c = jnp.sum(ppc_dx * npc[:, dy:dy + HS, :], axis=0)
            d = unroll(nss_c[dy:dy + HS, :] - 2.0 * cross_c) + pssp_c
            cross_r = jnp.sum(ppr_dx * npr[:, dy:dy + HS, :], axis=0)
            r = unroll(nss_r[dy:dy + HS, :] - 2.0 * cross_r) + pssp_r
            if best_d is None:
                best_d, best_raw = d, r
            else:
                upd = d < best_d
                best_d = jnp.where(upd, d, best_d)
                best_raw = jnp.where(upd, r, best_raw)
    d0, d1 = best_d[:, 0:HS], best_d[:, HP:HP + HS]
    r0, r1 = best_raw[:, 0:HS], best_raw[:, HP:HP + HS]
    final_raw = jnp.where(d1 < d0, r1, r0)
    out_ref[0] = jnp.full((8, 128), jnp.sum(final_raw), jnp.float32)


def kernel(pred, I):
    b = pred.shape[0]
    hp = H + 2
    xpred = jnp.pad(pred, ((0, 0), (0, 0), (1, 1), (1, 1)), mode='reflect')
    xi = jnp.pad(I, ((0, 0), (0, 0), (0, 0), (1, 1), (1, 1)), mode='reflect')
    w_op = jnp.asarray(_RESIZE_W)
    spy, spj = jnp.asarray(_SPY), jnp.asarray(_SPJ)
    sny, snj = jnp.asarray(_SNY), jnp.asarray(_SNJ)
    partial = pl.pallas_call(
        _fused_body,
        grid=(b,),
        in_specs=[
            pl.BlockSpec((1, C, hp, hp), lambda i: (i, 0, 0, 0)),
            pl.BlockSpec((1, 2, C, hp, hp), lambda i: (i, 0, 0, 0, 0)),
            pl.BlockSpec((HS, H), lambda i: (0, 0)),
            pl.BlockSpec(_SPY.shape, lambda i: (0, 0)),
            pl.BlockSpec(_SPJ.shape, lambda i: (0, 0, 0)),
            pl.BlockSpec(_SNY.shape, lambda i: (0, 0)),
            pl.BlockSpec(_SNJ.shape, lambda i: (0, 0, 0)),
        ],
        out_specs=pl.BlockSpec((1, 8, 128), lambda i: (i, 0, 0)),
        out_shape=jax.ShapeDtypeStruct((b, 8, 128), jnp.float32),
    )(xpred, xi, w_op, spy, spj, sny, snj)
    total = jnp.sum(partial[:, 0, 0])
    return total * (0.5 / (b * HS * HS * C1))


# speedup vs baseline: 1.3001x; 1.3001x over previous
import numpy as np
import jax
import jax.numpy as jnp
from jax.experimental import pallas as pl

KSIZE = 3
NSIZE = 7
H = 224
HS = 56
C = 3
C1 = C * KSIZE * KSIZE
PAD2 = NSIZE // 2

_HIGH = jax.lax.Precision.HIGHEST


def _resize_matrix(in_size, out_size):
    scale = out_size / in_size
    inv_scale = 1.0 / scale
    kernel_scale = max(inv_scale, 1.0)
    sample_f = (np.arange(out_size, dtype=np.float32) + 0.5) * inv_scale - 0.5
    x = np.abs(sample_f[None, :]
               - np.arange(in_size, dtype=np.float32)[:, None]) / kernel_scale
    x = x.astype(np.float32)
    w = (((1.5 * x - 2.5) * x * x + 1.0) * (x <= 1.0)
         + ((((-0.5 * x + 2.5) * x - 4.0) * x + 2.0)
            * ((x > 1.0) & (x < 2.0)))).astype(np.float32)
    total = w.sum(axis=0, keepdims=True)
    w = np.where(np.abs(total) > 1000 * np.finfo(np.float32).eps,
                 w / np.where(total != 0, total, 1), 0)
    w = np.where(((sample_f >= -0.5) & (sample_f <= in_size - 0.5))[None, :],
                 w, 0)
    return np.ascontiguousarray(w.T.astype(np.float32))


_RESIZE_W = _resize_matrix(H, HS)


def _refl(i, n):
    if i < 0:
        return -i
    if i >= n:
        return 2 * n - 2 - i
    return i


def _sel_matrices():
    spj = np.zeros((KSIZE, HS, HS), np.float32)
    for i in range(KSIZE):
        for y in range(HS):
            spj[i, y, _refl(y + i - 1, HS)] = 1.0
    HP = HS + 2 * PAD2
    snj = np.zeros((KSIZE, HP, HS), np.float32)
    for i in range(KSIZE):
        for Y in range(HP):
            snj[i, Y, _refl(_refl(Y - PAD2, HS) + i - 1, HS)] = 1.0
    spy = spj.reshape(KSIZE * HS, HS)
    sny = snj.reshape(KSIZE * HP, HS)
    return spy, spj, sny, snj


_SPY, _SPJ, _SNY, _SNJ = _sel_matrices()


def _cpad(x, p, axis):
    n = x.shape[axis]

    def sl(a, b):
        return tuple(slice(a, b) if d == axis else slice(None)
                     for d in range(x.ndim))

    parts = ([x[sl(t, t + 1)] for t in range(p, 0, -1)]
             + [x]
             + [x[sl(n - 1 - t, n - t)] for t in range(1, p + 1)])
    return jnp.concatenate(parts, axis=axis)


def _fused_body(xpred_ref, xi_ref, w_ref, spy_ref, spj_ref, sny_ref, snj_ref,
                out_ref):
    w = w_ref[...]

    def census(x):
        xp = _cpad(_cpad(x, 1, 1), 1, 2)
        acc = jnp.zeros((C, H, H), jnp.float32)
        for i in range(KSIZE):
            for j in range(KSIZE):
                acc = acc + jnp.tanh(xp[:, i:i + H, j:j + H] - x)
        return acc * (1.0 / (KSIZE * KSIZE))

    def resize(m):
        t1 = jax.lax.dot_general(m, w, (((1,), (1,)), ((), ())),
                                 precision=_HIGH)
        return jax.lax.dot_general(t1, w, (((1,), (1,)), ((), ())),
                                   precision=_HIGH)

    def patch_dots(r, sy_ref, sj_ref, rows):
        b1 = jax.lax.dot_general(r, sy_ref[...], (((1,), (1,)), ((), ())),
                                 precision=_HIGH)
        cs = []
        for j in range(KSIZE):
            cj = jax.lax.dot_general(b1, sj_ref[j], (((1,), (1,)), ((), ())),
                                     precision=_HIGH)
            cs.append(cj.reshape(C, KSIZE, rows, rows))
        return jnp.stack(cs, axis=2).reshape(C1, rows, rows)

    maps = [xpred_ref[0], xi_ref[0, 0], xi_ref[0, 1]]
    rz_ct = [resize(census(m)) for m in maps]
    rz_raw = [resize(m) for m in maps]

    HP2 = HS + 2 * PAD2
    pct = patch_dots(rz_ct[0], spy_ref, spj_ref, HS)
    praw = patch_dots(rz_raw[0], spy_ref, spj_ref, HS)
    nct = [patch_dots(rz_ct[k], sny_ref, snj_ref, HP2) for k in (1, 2)]
    nraw = [patch_dots(rz_raw[k], sny_ref, snj_ref, HP2) for k in (1, 2)]

    HP = HS + 2 * PAD2
    npc = jnp.concatenate(nct, axis=2)
    npr = jnp.concatenate(nraw, axis=2)
    z6 = jnp.zeros((C1, HS, HP - HS), jnp.float32)
    ppc = jnp.concatenate([pct, z6, pct, z6], axis=2)
    ppr = jnp.concatenate([praw, z6, praw, z6], axis=2)
    z6s = jnp.zeros((HS, HP - HS), jnp.float32)
    pss_c = jnp.sum(pct * pct, axis=0)
    pss_r = jnp.sum(praw * praw, axis=0)
    pssp_c = jnp.concatenate([pss_c, z6s, pss_c, z6s], axis=1)
    pssp_r = jnp.concatenate([pss_r, z6s, pss_r, z6s], axis=1)
    nss_c = jnp.sum(npc * npc, axis=0)
    nss_r = jnp.sum(npr * npr, axis=0)

    best_d = None
    best_raw = None
    for dx in range(NSIZE):
        ppc_dx = jnp.roll(ppc, dx, axis=2) if dx else ppc
        ppr_dx = jnp.roll(ppr, dx, axis=2) if dx else ppr
        def unroll(x):
            return jnp.roll(x, -dx, axis=1) if dx else x

        for dy in range(NSIZE):
            cross_c = jnp.sum(ppc_dx * npc[:, dy:dy + HS, :], axis=0)
            d = unroll(nss_c[dy:dy + HS, :] - 2.0 * cross_c) + pssp_c
            cross_r = jnp.sum(ppr_dx * npr[:, dy:dy + HS, :], axis=0)
            r = unroll(nss_r[dy:dy + HS, :] - 2.0 * cross_r) + pssp_r
            if best_d is None:
                best_d, best_raw = d, r
            else:
                upd = d < best_d
                best_d = jnp.where(upd, d, best_d)
                best_raw = jnp.where(upd, r, best_raw)
    d0, d1 = best_d[:, 0:HS], best_d[:, HP:HP + HS]
    r0, r1 = best_raw[:, 0:HS], best_raw[:, HP:HP + HS]
    final_raw = jnp.where(d1 < d0, r1, r0)
    out_ref[0] = jnp.full((8, 128), jnp.sum(final_raw), jnp.float32)


def kernel(pred, I):
    b = pred.shape[0]
    w_op = jnp.asarray(_RESIZE_W)
    spy, spj = jnp.asarray(_SPY), jnp.asarray(_SPJ)
    sny, snj = jnp.asarray(_SNY), jnp.asarray(_SNJ)
    partial = pl.pallas_call(
        _fused_body,
        grid=(b,),
        in_specs=[
            pl.BlockSpec((1, C, H, H), lambda i: (i, 0, 0, 0)),
            pl.BlockSpec((1, 2, C, H, H), lambda i: (i, 0, 0, 0, 0)),
            pl.BlockSpec((HS, H), lambda i: (0, 0)),
            pl.BlockSpec(_SPY.shape, lambda i: (0, 0)),
            pl.BlockSpec(_SPJ.shape, lambda i: (0, 0, 0)),
            pl.BlockSpec(_SNY.shape, lambda i: (0, 0)),
            pl.BlockSpec(_SNJ.shape, lambda i: (0, 0, 0)),
        ],
        out_specs=pl.BlockSpec((1, 8, 128), lambda i: (i, 0, 0)),
        out_shape=jax.ShapeDtypeStruct((b, 8, 128), jnp.float32),
    )(pred, I, w_op, spy, spj, sny, snj)
    total = jnp.sum(partial[:, 0, 0])
    return total * (0.5 / (b * HS * HS * C1))


# speedup vs baseline: 1.3290x; 1.0223x over previous
import numpy as np
import jax
import jax.numpy as jnp
from jax.experimental import pallas as pl

KSIZE = 3
NSIZE = 7
H = 224
HS = 56
C = 3
C1 = C * KSIZE * KSIZE
PAD2 = NSIZE // 2
HP = HS + 2 * PAD2

_HIGH = jax.lax.Precision.HIGHEST


def _resize_matrix(in_size, out_size):
    scale = out_size / in_size
    inv_scale = 1.0 / scale
    kernel_scale = max(inv_scale, 1.0)
    sample_f = (np.arange(out_size, dtype=np.float32) + 0.5) * inv_scale - 0.5
    x = np.abs(sample_f[None, :]
               - np.arange(in_size, dtype=np.float32)[:, None]) / kernel_scale
    x = x.astype(np.float32)
    w = (((1.5 * x - 2.5) * x * x + 1.0) * (x <= 1.0)
         + ((((-0.5 * x + 2.5) * x - 4.0) * x + 2.0)
            * ((x > 1.0) & (x < 2.0)))).astype(np.float32)
    total = w.sum(axis=0, keepdims=True)
    w = np.where(np.abs(total) > 1000 * np.finfo(np.float32).eps,
                 w / np.where(total != 0, total, 1), 0)
    w = np.where(((sample_f >= -0.5) & (sample_f <= in_size - 0.5))[None, :],
                 w, 0)
    return np.ascontiguousarray(w.T.astype(np.float32))


_RESIZE_W = _resize_matrix(H, HS)


def _refl(i, n):
    if i < 0:
        return -i
    if i >= n:
        return 2 * n - 2 - i
    return i


def _sel_matrices():
    spj = np.zeros((KSIZE, HS, HS), np.float32)
    for i in range(KSIZE):
        for y in range(HS):
            spj[i, y, _refl(y + i - 1, HS)] = 1.0
    HP = HS + 2 * PAD2
    snj = np.zeros((KSIZE, HP, HS), np.float32)
    for i in range(KSIZE):
        for Y in range(HP):
            snj[i, Y, _refl(_refl(Y - PAD2, HS) + i - 1, HS)] = 1.0
    spy = spj.reshape(KSIZE * HS, HS)
    sny = snj.reshape(KSIZE * HP, HS)
    spj2 = np.zeros((KSIZE, 2 * HP, 2 * HS), np.float32)
    snj2 = np.zeros((KSIZE, 2 * HP, 2 * HS), np.float32)
    for i in range(KSIZE):
        spj2[i, :HS, :HS] = spj[i]
        spj2[i, HP:HP + HS, HS:] = spj[i]
        snj2[i, :HP, :HS] = snj[i]
        snj2[i, HP:, HS:] = snj[i]
    return spy, sny, spj2, snj2


_SPY, _SNY, _SPJ2, _SNJ2 = _sel_matrices()


def _cpad(x, p, axis):
    n = x.shape[axis]

    def sl(a, b):
        return tuple(slice(a, b) if d == axis else slice(None)
                     for d in range(x.ndim))

    parts = ([x[sl(t, t + 1)] for t in range(p, 0, -1)]
             + [x]
             + [x[sl(n - 1 - t, n - t)] for t in range(1, p + 1)])
    return jnp.concatenate(parts, axis=axis)


def _fused_body(xpred_ref, xi_ref, w_ref, spy_ref, spj_ref, sny_ref, snj_ref,
                out_ref):
    w = w_ref[...]

    def census(x):
        xp = _cpad(_cpad(x, 1, 1), 1, 2)
        acc = jnp.zeros((3 * C, H, H), jnp.float32)
        for i in range(KSIZE):
            for j in range(KSIZE):
                acc = acc + jnp.tanh(xp[:, i:i + H, j:j + H] - x)
        return acc * (1.0 / (KSIZE * KSIZE))

    def resize(m):
        t1 = jax.lax.dot_general(m, w, (((1,), (1,)), ((), ())),
                                 precision=_HIGH)
        return jax.lax.dot_general(t1, w, (((1,), (1,)), ((), ())),
                                   precision=_HIGH)

    def patch_packed(rr, sy_ref, sj2_ref, rows):
        b1 = jax.lax.dot_general(rr, sy_ref[...], (((1,), (1,)), ((), ())),
                                 precision=_HIGH)
        cs = []
        for j in range(KSIZE):
            cj = jax.lax.dot_general(b1, sj2_ref[j], (((1,), (1,)), ((), ())),
                                     precision=_HIGH)
            cs.append(cj.reshape(C, KSIZE, rows, 2 * HP))
        return jnp.stack(cs, axis=2).reshape(C1, rows, 2 * HP)

    x9 = jnp.concatenate([xpred_ref[0], xi_ref[0, 0], xi_ref[0, 1]], axis=0)
    rz_ct = resize(census(x9))
    rz_raw = resize(x9)

    rp_ct = jnp.concatenate([rz_ct[0:C], rz_ct[0:C]], axis=2)
    rp_raw = jnp.concatenate([rz_raw[0:C], rz_raw[0:C]], axis=2)
    rn_ct = jnp.concatenate([rz_ct[C:2 * C], rz_ct[2 * C:]], axis=2)
    rn_raw = jnp.concatenate([rz_raw[C:2 * C], rz_raw[2 * C:]], axis=2)

    ppc = patch_packed(rp_ct, spy_ref, spj_ref, HS)
    ppr = patch_packed(rp_raw, spy_ref, spj_ref, HS)
    npc = patch_packed(rn_ct, sny_ref, snj_ref, HP)
    npr = patch_packed(rn_raw, sny_ref, snj_ref, HP)
    pssp_c = jnp.sum(ppc * ppc, axis=0)
    pssp_r = jnp.sum(ppr * ppr, axis=0)
    nss_c = jnp.sum(npc * npc, axis=0)
    nss_r = jnp.sum(npr * npr, axis=0)

    best_d = None
    best_raw = None
    for dx in range(NSIZE):
        ppc_dx = jnp.roll(ppc, dx, axis=2) if dx else ppc
        ppr_dx = jnp.roll(ppr, dx, axis=2) if dx else ppr
        def unroll(x):
            return jnp.roll(x, -dx, axis=1) if dx else x

        for dy in range(NSIZE):
            cross_c = jnp.sum(ppc_dx * npc[:, dy:dy + HS, :], axis=0)
            d = unroll(nss_c[dy:dy + HS, :] - 2.0 * cross_c) + pssp_c
            cross_r = jnp.sum(ppr_dx * npr[:, dy:dy + HS, :], axis=0)
            r = unroll(nss_r[dy:dy + HS, :] - 2.0 * cross_r) + pssp_r
            if best_d is None:
                best_d, best_raw = d, r
            else:
                upd = d < best_d
                best_d = jnp.where(upd, d, best_d)
                best_raw = jnp.where(upd, r, best_raw)
    d0, d1 = best_d[:, 0:HS], best_d[:, HP:HP + HS]
    r0, r1 = best_raw[:, 0:HS], best_raw[:, HP:HP + HS]
    final_raw = jnp.where(d1 < d0, r1, r0)
    out_ref[0] = jnp.full((8, 128), jnp.sum(final_raw), jnp.float32)


def kernel(pred, I):
    b = pred.shape[0]
    w_op = jnp.asarray(_RESIZE_W)
    spy, spj2 = jnp.asarray(_SPY), jnp.asarray(_SPJ2)
    sny, snj2 = jnp.asarray(_SNY), jnp.asarray(_SNJ2)
    partial = pl.pallas_call(
        _fused_body,
        grid=(b,),
        in_specs=[
            pl.BlockSpec((1, C, H, H), lambda i: (i, 0, 0, 0)),
            pl.BlockSpec((1, 2, C, H, H), lambda i: (i, 0, 0, 0, 0)),
            pl.BlockSpec((HS, H), lambda i: (0, 0)),
            pl.BlockSpec(_SPY.shape, lambda i: (0, 0)),
            pl.BlockSpec(_SPJ2.shape, lambda i: (0, 0, 0)),
            pl.BlockSpec(_SNY.shape, lambda i: (0, 0)),
            pl.BlockSpec(_SNJ2.shape, lambda i: (0, 0, 0)),
        ],
        out_specs=pl.BlockSpec((1, 8, 128), lambda i: (i, 0, 0)),
        out_shape=jax.ShapeDtypeStruct((b, 8, 128), jnp.float32),
    )(pred, I, w_op, spy, spj2, sny, snj2)
    total = jnp.sum(partial[:, 0, 0])
    return total * (0.5 / (b * HS * HS * C1))


# speedup vs baseline: 1.3361x; 1.0053x over previous
import numpy as np
import jax
import jax.numpy as jnp
from jax.experimental import pallas as pl

KSIZE = 3
NSIZE = 7
H = 224
HS = 56
C = 3
C1 = C * KSIZE * KSIZE
PAD2 = NSIZE // 2
HP = HS + 2 * PAD2

_HIGH = jax.lax.Precision.HIGHEST


def _resize_matrix(in_size, out_size):
    scale = out_size / in_size
    inv_scale = 1.0 / scale
    kernel_scale = max(inv_scale, 1.0)
    sample_f = (np.arange(out_size, dtype=np.float32) + 0.5) * inv_scale - 0.5
    x = np.abs(sample_f[None, :]
               - np.arange(in_size, dtype=np.float32)[:, None]) / kernel_scale
    x = x.astype(np.float32)
    w = (((1.5 * x - 2.5) * x * x + 1.0) * (x <= 1.0)
         + ((((-0.5 * x + 2.5) * x - 4.0) * x + 2.0)
            * ((x > 1.0) & (x < 2.0)))).astype(np.float32)
    total = w.sum(axis=0, keepdims=True)
    w = np.where(np.abs(total) > 1000 * np.finfo(np.float32).eps,
                 w / np.where(total != 0, total, 1), 0)
    w = np.where(((sample_f >= -0.5) & (sample_f <= in_size - 0.5))[None, :],
                 w, 0)
    return np.ascontiguousarray(w.T.astype(np.float32))


_RESIZE_W = _resize_matrix(H, HS)


def _refl(i, n):
    if i < 0:
        return -i
    if i >= n:
        return 2 * n - 2 - i
    return i


def _sel_matrices():
    spj = np.zeros((KSIZE, HS, HS), np.float32)
    for i in range(KSIZE):
        for y in range(HS):
            spj[i, y, _refl(y + i - 1, HS)] = 1.0
    HP = HS + 2 * PAD2
    snj = np.zeros((KSIZE, HP, HS), np.float32)
    for i in range(KSIZE):
        for Y in range(HP):
            snj[i, Y, _refl(_refl(Y - PAD2, HS) + i - 1, HS)] = 1.0
    spy = spj.reshape(KSIZE * HS, HS)
    sny = snj.reshape(KSIZE * HP, HS)
    spj2 = np.zeros((KSIZE, 2 * HP, 2 * HS), np.float32)
    snj2 = np.zeros((KSIZE, 2 * HP, 2 * HS), np.float32)
    for i in range(KSIZE):
        spj2[i, :HS, :HS] = spj[i]
        spj2[i, HP:HP + HS, HS:] = spj[i]
        snj2[i, :HP, :HS] = snj[i]
        snj2[i, HP:, HS:] = snj[i]
    return spy, sny, spj2, snj2


_SPY, _SNY, _SPJ2, _SNJ2 = _sel_matrices()


def _cpad(x, p, axis):
    n = x.shape[axis]

    def sl(a, b):
        return tuple(slice(a, b) if d == axis else slice(None)
                     for d in range(x.ndim))

    parts = ([x[sl(t, t + 1)] for t in range(p, 0, -1)]
             + [x]
             + [x[sl(n - 1 - t, n - t)] for t in range(1, p + 1)])
    return jnp.concatenate(parts, axis=axis)


def _fused_body(xpred_ref, xi_ref, w_ref, spy_ref, spj_ref, sny_ref, snj_ref,
                out_ref):
    w = w_ref[...]

    def census(x):
        xp = _cpad(_cpad(x, 1, 1), 1, 2)
        acc = jnp.zeros((3 * C, H, H), jnp.float32)
        for i in range(KSIZE):
            for j in range(KSIZE):
                acc = acc + jnp.tanh(xp[:, i:i + H, j:j + H] - x)
        return acc * (1.0 / (KSIZE * KSIZE))

    def resize(m):
        t1 = jax.lax.dot_general(m, w, (((1,), (1,)), ((), ())),
                                 precision=_HIGH)
        return jax.lax.dot_general(t1, w, (((1,), (1,)), ((), ())),
                                   precision=_HIGH)

    def patch_packed(rr, sy_ref, sj2_ref, rows):
        b1 = jax.lax.dot_general(rr, sy_ref[...], (((1,), (1,)), ((), ())),
                                 precision=_HIGH)
        cs = []
        for j in range(KSIZE):
            cj = jax.lax.dot_general(b1, sj2_ref[j], (((1,), (1,)), ((), ())),
                                     precision=_HIGH)
            cs.append(cj.reshape(C, KSIZE, rows, 2 * HP))
        return jnp.stack(cs, axis=0).reshape(C1, rows, 2 * HP)

    x9 = jnp.concatenate([xpred_ref[0], xi_ref[0, 0], xi_ref[0, 1]], axis=0)
    rz_ct = resize(census(x9))
    rz_raw = resize(x9)

    rp_ct = jnp.concatenate([rz_ct[0:C], rz_ct[0:C]], axis=2)
    rp_raw = jnp.concatenate([rz_raw[0:C], rz_raw[0:C]], axis=2)
    rn_ct = jnp.concatenate([rz_ct[C:2 * C], rz_ct[2 * C:]], axis=2)
    rn_raw = jnp.concatenate([rz_raw[C:2 * C], rz_raw[2 * C:]], axis=2)

    ppc = patch_packed(rp_ct, spy_ref, spj_ref, HS)
    ppr = patch_packed(rp_raw, spy_ref, spj_ref, HS)
    npc = patch_packed(rn_ct, sny_ref, snj_ref, HP)
    npr = patch_packed(rn_raw, sny_ref, snj_ref, HP)
    pss_r = jnp.sum(ppr[:, :, 0:HS] * ppr[:, :, 0:HS], axis=0)
    nssh_c = 0.5 * jnp.sum(npc * npc, axis=0)
    nssh_r = 0.5 * jnp.sum(npr * npr, axis=0)

    best_d = None
    best_raw = None
    for dx in range(NSIZE):
        ppc_dx = jnp.roll(ppc, dx, axis=2) if dx else ppc
        ppr_dx = jnp.roll(ppr, dx, axis=2) if dx else ppr
        def unroll(x):
            return jnp.roll(x, -dx, axis=1) if dx else x

        for dy in range(NSIZE):
            cross_c = jnp.sum(ppc_dx * npc[:, dy:dy + HS, :], axis=0)
            d = unroll(nssh_c[dy:dy + HS, :] - cross_c)
            cross_r = jnp.sum(ppr_dx * npr[:, dy:dy + HS, :], axis=0)
            r = unroll(nssh_r[dy:dy + HS, :] - cross_r)
            if best_d is None:
                best_d, best_raw = d, r
            else:
                upd = d < best_d
                best_d = jnp.where(upd, d, best_d)
                best_raw = jnp.where(upd, r, best_raw)
    d0, d1 = best_d[:, 0:HS], best_d[:, HP:HP + HS]
    r0, r1 = best_raw[:, 0:HS], best_raw[:, HP:HP + HS]
    final_raw = pss_r + 2.0 * jnp.where(d1 < d0, r1, r0)
    out_ref[0] = jnp.full((8, 128), jnp.sum(final_raw), jnp.float32)


def kernel(pred, I):
    b = pred.shape[0]
    w_op = jnp.asarray(_RESIZE_W)
    spy, spj2 = jnp.asarray(_SPY), jnp.asarray(_SPJ2)
    sny, snj2 = jnp.asarray(_SNY), jnp.asarray(_SNJ2)
    partial = pl.pallas_call(
        _fused_body,
        grid=(b,),
        in_specs=[
            pl.BlockSpec((1, C, H, H), lambda i: (i, 0, 0, 0)),
            pl.BlockSpec((1, 2, C, H, H), lambda i: (i, 0, 0, 0, 0)),
            pl.BlockSpec((HS, H), lambda i: (0, 0)),
            pl.BlockSpec(_SPY.shape, lambda i: (0, 0)),
            pl.BlockSpec(_SPJ2.shape, lambda i: (0, 0, 0)),
            pl.BlockSpec(_SNY.shape, lambda i: (0, 0)),
            pl.BlockSpec(_SNJ2.shape, lambda i: (0, 0, 0)),
        ],
        out_specs=pl.BlockSpec((1, 8, 128), lambda i: (i, 0, 0)),
        out_shape=jax.ShapeDtypeStruct((b, 8, 128), jnp.float32),
    )(pred, I, w_op, spy, spj2, sny, snj2)
    total = jnp.sum(partial[:, 0, 0])
    return total * (0.5 / (b * HS * HS * C1))


# speedup vs baseline: 1.4319x; 1.0717x over previous
import numpy as np
import jax
import jax.numpy as jnp
from jax.experimental import pallas as pl

KSIZE = 3
NSIZE = 7
H = 224
HS = 56
C = 3
C1 = C * KSIZE * KSIZE
PAD2 = NSIZE // 2
HP = HS + 2 * PAD2

_HIGH = jax.lax.Precision.HIGHEST


def _resize_matrix(in_size, out_size):
    scale = out_size / in_size
    inv_scale = 1.0 / scale
    kernel_scale = max(inv_scale, 1.0)
    sample_f = (np.arange(out_size, dtype=np.float32) + 0.5) * inv_scale - 0.5
    x = np.abs(sample_f[None, :]
               - np.arange(in_size, dtype=np.float32)[:, None]) / kernel_scale
    x = x.astype(np.float32)
    w = (((1.5 * x - 2.5) * x * x + 1.0) * (x <= 1.0)
         + ((((-0.5 * x + 2.5) * x - 4.0) * x + 2.0)
            * ((x > 1.0) & (x < 2.0)))).astype(np.float32)
    total = w.sum(axis=0, keepdims=True)
    w = np.where(np.abs(total) > 1000 * np.finfo(np.float32).eps,
                 w / np.where(total != 0, total, 1), 0)
    w = np.where(((sample_f >= -0.5) & (sample_f <= in_size - 0.5))[None, :],
                 w, 0)
    return np.ascontiguousarray(w.T.astype(np.float32))


_RESIZE_W = _resize_matrix(H, HS)


def _refl(i, n):
    if i < 0:
        return -i
    if i >= n:
        return 2 * n - 2 - i
    return i


def _sel_matrices():
    spj = np.zeros((KSIZE, HS, HS), np.float32)
    for i in range(KSIZE):
        for y in range(HS):
            spj[i, y, _refl(y + i - 1, HS)] = 1.0
    HP = HS + 2 * PAD2
    snj = np.zeros((KSIZE, HP, HS), np.float32)
    for i in range(KSIZE):
        for Y in range(HP):
            snj[i, Y, _refl(_refl(Y - PAD2, HS) + i - 1, HS)] = 1.0
    spy = spj.reshape(KSIZE * HS, HS)
    sny = snj.reshape(KSIZE * HP, HS)
    spj2 = np.zeros((KSIZE, 2 * HP, 2 * HS), np.float32)
    snj2 = np.zeros((KSIZE, 2 * HP, 2 * HS), np.float32)
    for i in range(KSIZE):
        spj2[i, :HS, :HS] = spj[i]
        spj2[i, HP:HP + HS, HS:] = spj[i]
        snj2[i, :HP, :HS] = snj[i]
        snj2[i, HP:, HS:] = snj[i]
    return spy, sny, spj2, snj2


_SPY, _SNY, _SPJ2, _SNJ2 = _sel_matrices()


def _cpad(x, p, axis):
    n = x.shape[axis]

    def sl(a, b):
        return tuple(slice(a, b) if d == axis else slice(None)
                     for d in range(x.ndim))

    parts = ([x[sl(t, t + 1)] for t in range(p, 0, -1)]
             + [x]
             + [x[sl(n - 1 - t, n - t)] for t in range(1, p + 1)])
    return jnp.concatenate(parts, axis=axis)


def _fused_body(xpred_ref, xi_ref, w_ref, spy_ref, spj_ref, sny_ref, snj_ref,
                out_ref):
    w = w_ref[...]

    def census(x):
        xp = _cpad(_cpad(x, 1, 1), 1, 2)
        acc = jnp.zeros((3 * C, H, H), jnp.float32)
        for i in range(KSIZE):
            for j in range(KSIZE):
                acc = acc + jnp.tanh(xp[:, i:i + H, j:j + H] - x)
        return acc * (1.0 / (KSIZE * KSIZE))

    def resize(m):
        t1 = jax.lax.dot_general(m, w, (((1,), (1,)), ((), ())),
                                 precision=_HIGH)
        return jax.lax.dot_general(t1, w, (((1,), (1,)), ((), ())),
                                   precision=_HIGH)

    def patch_packed(rr, sy_ref, sj2_ref, rows):
        b1 = jax.lax.dot_general(rr, sy_ref[...], (((1,), (1,)), ((), ())),
                                 precision=_HIGH)
        cs = []
        for j in range(KSIZE):
            cj = jax.lax.dot_general(b1, sj2_ref[j], (((1,), (1,)), ((), ())),
                                     precision=_HIGH)
            cs.append(cj.reshape(C, KSIZE, rows, 2 * HP))
        return jnp.stack(cs, axis=0).reshape(C1, rows, 2 * HP)

    x9 = jnp.concatenate([xpred_ref[0], xi_ref[0, 0], xi_ref[0, 1]], axis=0)
    rz_ct = resize(census(x9))
    rz_raw = resize(x9)

    rp_ct = jnp.concatenate([rz_ct[0:C], rz_ct[0:C]], axis=2)
    rp_raw = jnp.concatenate([rz_raw[0:C], rz_raw[0:C]], axis=2)
    rn_ct = jnp.concatenate([rz_ct[C:2 * C], rz_ct[2 * C:]], axis=2)
    rn_raw = jnp.concatenate([rz_raw[C:2 * C], rz_raw[2 * C:]], axis=2)

    ppc = patch_packed(rp_ct, spy_ref, spj_ref, HS)
    ppr = patch_packed(rp_raw, spy_ref, spj_ref, HS)
    npc = patch_packed(rn_ct, sny_ref, snj_ref, HP)
    npr = patch_packed(rn_raw, sny_ref, snj_ref, HP)
    pss_r = jnp.sum(ppr[:, :, 0:HS] * ppr[:, :, 0:HS], axis=0)
    nssh_c = 0.5 * jnp.sum(npc * npc, axis=0)
    nssh_r = 0.5 * jnp.sum(npr * npr, axis=0)

    best_d = None
    best_raw = None
    for dx in range(NSIZE):
        ppc_dx = jnp.roll(ppc, dx, axis=2) if dx else ppc
        ppr_dx = jnp.roll(ppr, dx, axis=2) if dx else ppr
        bd = None
        br = None
        for dy in range(NSIZE):
            cross_c = jnp.sum(ppc_dx * npc[:, dy:dy + HS, :], axis=0)
            d = nssh_c[dy:dy + HS, :] - cross_c
            cross_r = jnp.sum(ppr_dx * npr[:, dy:dy + HS, :], axis=0)
            r = nssh_r[dy:dy + HS, :] - cross_r
            if bd is None:
                bd, br = d, r
            else:
                upd = d < bd
                bd = jnp.where(upd, d, bd)
                br = jnp.where(upd, r, br)
        if dx:
            bd = jnp.roll(bd, -dx, axis=1)
            br = jnp.roll(br, -dx, axis=1)
        if best_d is None:
            best_d, best_raw = bd, br
        else:
            upd = bd < best_d
            best_d = jnp.where(upd, bd, best_d)
            best_raw = jnp.where(upd, br, best_raw)
    d0, d1 = best_d[:, 0:HS], best_d[:, HP:HP + HS]
    r0, r1 = best_raw[:, 0:HS], best_raw[:, HP:HP + HS]
    final_raw = pss_r + 2.0 * jnp.where(d1 < d0, r1, r0)
    out_ref[0] = jnp.full((8, 128), jnp.sum(final_raw), jnp.float32)


def kernel(pred, I):
    b = pred.shape[0]
    w_op = jnp.asarray(_RESIZE_W)
    spy, spj2 = jnp.asarray(_SPY), jnp.asarray(_SPJ2)
    sny, snj2 = jnp.asarray(_SNY), jnp.asarray(_SNJ2)
    partial = pl.pallas_call(
        _fused_body,
        grid=(b,),
        in_specs=[
            pl.BlockSpec((1, C, H, H), lambda i: (i, 0, 0, 0)),
            pl.BlockSpec((1, 2, C, H, H), lambda i: (i, 0, 0, 0, 0)),
            pl.BlockSpec((HS, H), lambda i: (0, 0)),
            pl.BlockSpec(_SPY.shape, lambda i: (0, 0)),
            pl.BlockSpec(_SPJ2.shape, lambda i: (0, 0, 0)),
            pl.BlockSpec(_SNY.shape, lambda i: (0, 0)),
            pl.BlockSpec(_SNJ2.shape, lambda i: (0, 0, 0)),
        ],
        out_specs=pl.BlockSpec((1, 8, 128), lambda i: (i, 0, 0)),
        out_shape=jax.ShapeDtypeStruct((b, 8, 128), jnp.float32),
    )(pred, I, w_op, spy, spj2, sny, snj2)
    total = jnp.sum(partial[:, 0, 0])
    return total * (0.5 / (b * HS * HS * C1))
